# Initial kernel scaffold; baseline (speedup 1.0000x reference)
import jax
import jax.numpy as jnp
from jax.experimental import pallas as pl


def kernel(last_inv, cur_inv, last_equ, cur_equ, upsampling, W_last_equ, W_cur_equ, W_mlp, b_mlp, gamma, beta):
    raise NotImplementedError("write your pallas kernel here")


# speedup vs baseline: 1.3600x; 1.3600x over previous
import functools

import jax
import jax.numpy as jnp
from jax import lax
from jax.experimental import pallas as pl
from jax.experimental.pallas import tpu as pltpu
from jax.experimental.pallas import tpu_sc as plsc


def _coarse_body(li_ref, le_ref, w1_ref, wle_ref, out_ref):
    nb = le_ref.shape[1]
    oc = w1_ref.shape[1]
    ec = wle_ref.shape[1]
    out_ref[:, :oc] = jnp.dot(li_ref[...], w1_ref[...],
                              preferred_element_type=jnp.float32)
    for b in range(nb):
        out_ref[:, oc + ec * b: oc + ec * (b + 1)] = jnp.dot(
            le_ref[:, b, :], wle_ref[...], preferred_element_type=jnp.float32)


def _fine_body(ce_ref, ci_ref, g_ref, wce_ref, w2_ref, w3_ref,
               b_ref, gam_ref, bet_ref, out_ref):
    f = ce_ref.shape[0]
    nb = ce_ref.shape[1]
    ec = wce_ref.shape[1]
    oc = w2_ref.shape[1]
    acc = g_ref[:, :oc] + jnp.dot(ci_ref[...], w2_ref[...],
                                  preferred_element_type=jnp.float32)
    equ = None
    for b in range(nb):
        ceb = jnp.dot(ce_ref[:, b, :], wce_ref[...],
                      preferred_element_type=jnp.float32)
        prod = ceb * g_ref[:, oc + ec * b: oc + ec * (b + 1)]
        equ = prod if equ is None else equ + prod
    equ = equ * (1.0 / nb)
    acc = acc + jnp.dot(equ, w3_ref[...], preferred_element_type=jnp.float32)
    acc = acc + b_ref[...]
    groups = 32
    xg = acc.reshape(f, groups, oc // groups)
    mean = jnp.mean(xg, axis=-1, keepdims=True)
    var = jnp.mean(xg * xg, axis=-1, keepdims=True) - mean * mean
    xn = (xg - mean) * lax.rsqrt(var + 1e-5)
    x = xn.reshape(f, oc) * gam_ref[...] + bet_ref[...]
    out_ref[...] = jnp.where(x >= 0, x, 0.1 * x)


def _make_sc_gather(n_rows, d, dtype, chunk):
    info = plsc.get_sparse_core_info()
    nc, ns = info.num_cores, info.num_subcores
    nw = nc * ns
    b_per_w = n_rows // nw
    nch = b_per_w // chunk
    mesh = plsc.VectorSubcoreMesh(core_axis_name="c", subcore_axis_name="s")

    @functools.partial(
        pl.kernel, mesh=mesh,
        out_type=jax.ShapeDtypeStruct((n_rows, d), dtype),
        scratch_types=[
            pltpu.VMEM((chunk,), jnp.int32),
            pltpu.VMEM((chunk, d), dtype),
            pltpu.SemaphoreType.DMA,
        ],
    )
    def gk(table_hbm, idx_hbm, out_hbm, idx_v, rows_v, sem):
        wid = lax.axis_index("s") * nc + lax.axis_index("c")
        base = pl.multiple_of(wid * b_per_w, 8)

        def body(i, carry):
            off = pl.multiple_of(base + i * chunk, 8)
            pltpu.sync_copy(idx_hbm.at[pl.ds(off, chunk)], idx_v)
            pltpu.async_copy(table_hbm.at[idx_v], rows_v, sem).wait()
            pltpu.sync_copy(rows_v, out_hbm.at[pl.ds(off, chunk)])
            return carry

        lax.fori_loop(0, nch, body, 0)

    return gk


def kernel(last_inv, cur_inv, last_equ, cur_equ, upsampling,
           W_last_equ, W_cur_equ, W_mlp, b_mlp, gamma, beta):
    n_c, inv_l = last_inv.shape
    n_f, inv_c = cur_inv.shape
    nb, equ_l = last_equ.shape[1], last_equ.shape[2]
    equ_c = cur_equ.shape[2]
    out_c = W_mlp.shape[1]
    d = out_c + nb * equ_c

    W1 = W_mlp[:inv_l]
    W2 = W_mlp[inv_l:inv_l + inv_c]
    W3 = W_mlp[inv_l + inv_c:]

    bc = 512
    table = pl.pallas_call(
        _coarse_body,
        grid=(pl.cdiv(n_c, bc),),
        in_specs=[
            pl.BlockSpec((bc, inv_l), lambda i: (i, 0)),
            pl.BlockSpec((bc, nb, equ_l), lambda i: (i, 0, 0)),
            pl.BlockSpec((inv_l, out_c), lambda i: (0, 0)),
            pl.BlockSpec((equ_l, equ_c), lambda i: (0, 0)),
        ],
        out_specs=pl.BlockSpec((bc, d), lambda i: (i, 0)),
        out_shape=jax.ShapeDtypeStruct((n_c, d), jnp.float32),
    )(last_inv, last_equ, W1, W_last_equ)

    chunk = 56
    align = 32 * chunk
    n_pad = ((n_f + align - 1) // align) * align
    idx = upsampling[:, 0].astype(jnp.int32)
    idx = jnp.concatenate(
        [idx, jnp.zeros((n_pad - n_f,), dtype=jnp.int32)])
    gathered = _make_sc_gather(n_pad, d, jnp.float32, chunk)(table, idx)

    fb = 512
    out = pl.pallas_call(
        _fine_body,
        grid=(pl.cdiv(n_f, fb),),
        in_specs=[
            pl.BlockSpec((fb, nb, equ_c), lambda i: (i, 0, 0)),
            pl.BlockSpec((fb, inv_c), lambda i: (i, 0)),
            pl.BlockSpec((fb, d), lambda i: (i, 0)),
            pl.BlockSpec((equ_c, equ_c), lambda i: (0, 0)),
            pl.BlockSpec((inv_c, out_c), lambda i: (0, 0)),
            pl.BlockSpec((equ_c, out_c), lambda i: (0, 0)),
            pl.BlockSpec((1, out_c), lambda i: (0, 0)),
            pl.BlockSpec((1, out_c), lambda i: (0, 0)),
            pl.BlockSpec((1, out_c), lambda i: (0, 0)),
        ],
        out_specs=pl.BlockSpec((fb, out_c), lambda i: (i, 0)),
        out_shape=jax.ShapeDtypeStruct((n_f, out_c), jnp.float32),
    )(cur_equ, cur_inv, gathered, W_cur_equ, W2, W3,
      b_mlp.reshape(1, out_c), gamma.reshape(1, out_c), beta.reshape(1, out_c))
    return out
